# baseline (speedup 1.0000x reference)
import functools

import jax
import jax.numpy as jnp
from jax.experimental import pallas as pl
from jax.experimental.pallas import tpu as pltpu

H = 2048
E = 8
DFF = 2048
SH = 2048
T = 2048
TM = 256
KC = 512
NEG_INF = -1e30


def _silu(v):
    return v * (1.0 / (1.0 + jnp.exp(-v)))


def _dot_t(a, b):
    return jax.lax.dot_general(a, b, (((1,), (1,)), ((), ())),
                               preferred_element_type=jnp.float32)


def _shared_router_kernel(x_ref, gu_g_ref, gu_u_ref, down_ref, gate_ref,
                          sh_out_ref, wfull_ref, acc_ref):
    c = pl.program_id(1)

    x = x_ref[...]

    @pl.when(c == 0)
    def _():
        acc_ref[...] = jnp.zeros_like(acc_ref)
        logits = _dot_t(x, gate_ref[...])
        col = jax.lax.broadcasted_iota(jnp.int32, logits.shape, 1)
        m1 = jnp.max(logits, axis=1, keepdims=True)
        a1 = jnp.min(jnp.where(logits == m1, col, E), axis=1, keepdims=True)
        mask1 = col == a1
        l2 = jnp.where(mask1, NEG_INF, logits)
        m2 = jnp.max(l2, axis=1, keepdims=True)
        a2 = jnp.min(jnp.where(l2 == m2, col, E), axis=1, keepdims=True)
        mask2 = col == a2
        w1 = 1.0 / (1.0 + jnp.exp(m2 - m1))
        w2 = 1.0 - w1
        wfull_ref[...] = (jnp.where(mask1, w1, 0.0)
                          + jnp.where(mask2, w2, 0.0)).astype(jnp.float32)

    g = _dot_t(x, gu_g_ref[...])
    u = _dot_t(x, gu_u_ref[...])
    h = _silu(g) * u
    acc_ref[...] += _dot_t(h, down_ref[...])

    @pl.when(c == pl.num_programs(1) - 1)
    def _():
        sh_out_ref[...] = acc_ref[...]


def _moe_dense_kernel(x_ref, wfull_ref, wg_ref, wu_ref, wd_ref, sh_ref,
                      out_ref, acc_ref):
    e = pl.program_id(1)
    c = pl.program_id(2)

    @pl.when(jnp.logical_and(e == 0, c == 0))
    def _():
        acc_ref[...] = jnp.zeros_like(acc_ref)

    x = x_ref[...]
    g = _dot_t(x, wg_ref[0])
    u = _dot_t(x, wu_ref[0])
    col = jax.lax.broadcasted_iota(jnp.int32, wfull_ref.shape, 1)
    w_e = jnp.sum(jnp.where(col == e, wfull_ref[...], 0.0), axis=1,
                  keepdims=True)
    h = (_silu(g) * u) * w_e
    acc_ref[...] += _dot_t(h, wd_ref[0])

    @pl.when(jnp.logical_and(e == pl.num_programs(1) - 1,
                             c == pl.num_programs(2) - 1))
    def _():
        out_ref[...] = acc_ref[...] + sh_ref[...]


def kernel(hidden_states, gate_w, w_gate, w_up, w_down, shared_gate_up_w,
           shared_down_w):
    orig_shape = hidden_states.shape
    x = hidden_states.reshape(-1, H)

    sh_out, wfull = pl.pallas_call(
        _shared_router_kernel,
        grid=(T // TM, SH // KC),
        in_specs=[
            pl.BlockSpec((TM, H), lambda i, c: (i, 0)),
            pl.BlockSpec((KC, H), lambda i, c: (c, 0)),
            pl.BlockSpec((KC, H), lambda i, c: (SH // KC + c, 0)),
            pl.BlockSpec((H, KC), lambda i, c: (0, c)),
            pl.BlockSpec((E, H), lambda i, c: (0, 0)),
        ],
        out_specs=[
            pl.BlockSpec((TM, H), lambda i, c: (i, 0)),
            pl.BlockSpec((TM, E), lambda i, c: (i, 0)),
        ],
        out_shape=[
            jax.ShapeDtypeStruct((T, H), jnp.float32),
            jax.ShapeDtypeStruct((T, E), jnp.float32),
        ],
        scratch_shapes=[pltpu.VMEM((TM, H), jnp.float32)],
    )(x, shared_gate_up_w, shared_gate_up_w, shared_down_w, gate_w)

    out = pl.pallas_call(
        _moe_dense_kernel,
        grid=(T // TM, E, DFF // KC),
        in_specs=[
            pl.BlockSpec((TM, H), lambda i, e, c: (i, 0)),
            pl.BlockSpec((TM, E), lambda i, e, c: (i, 0)),
            pl.BlockSpec((1, KC, H), lambda i, e, c: (e, c, 0)),
            pl.BlockSpec((1, KC, H), lambda i, e, c: (e, c, 0)),
            pl.BlockSpec((1, H, KC), lambda i, e, c: (e, 0, c)),
            pl.BlockSpec((TM, H), lambda i, e, c: (i, 0)),
        ],
        out_specs=pl.BlockSpec((TM, H), lambda i, e, c: (i, 0)),
        out_shape=jax.ShapeDtypeStruct((T, H), jnp.float32),
        scratch_shapes=[pltpu.VMEM((TM, H), jnp.float32)],
    )(x, wfull, w_gate, w_up, w_down, sh_out)

    return out.reshape(orig_shape)


# speedup vs baseline: 1.6770x; 1.6770x over previous
import functools

import jax
import jax.numpy as jnp
from jax.experimental import pallas as pl
from jax.experimental.pallas import tpu as pltpu
from jax.experimental.pallas import tpu_sc as plsc

H = 2048
E = 8
DFF = 2048
SH = 2048
T = 2048
TOPK = 2

TM = 256
KC = 512
TMG = 256
L = ((T * TOPK + E * (TMG - 1)) + TMG - 1) // TMG * TMG
NTILES = L // TMG
NEG_INF = -1e30

NW = 32
TPW = T // NW


def _silu(v):
    return v * (1.0 / (1.0 + jnp.exp(-v)))


def _dot_t(a, b):
    return jax.lax.dot_general(a, b, (((1,), (1,)), ((), ())),
                               preferred_element_type=jnp.float32)


def _top2(logits):
    col = jax.lax.broadcasted_iota(jnp.int32, logits.shape, 1)
    m1 = jnp.max(logits, axis=1, keepdims=True)
    a1 = jnp.min(jnp.where(logits == m1, col, E), axis=1, keepdims=True)
    mask1 = col == a1
    l2 = jnp.where(mask1, NEG_INF, logits)
    m2 = jnp.max(l2, axis=1, keepdims=True)
    a2 = jnp.min(jnp.where(l2 == m2, col, E), axis=1, keepdims=True)
    mask2 = col == a2
    w1 = 1.0 / (1.0 + jnp.exp(m2 - m1))
    w2 = 1.0 - w1
    return mask1, mask2, w1, w2


def _router_kernel(x_ref, gate_ref, pos_ref, w_ref, meta_ref, st_ref):
    p = pl.program_id(0)
    i = pl.program_id(1)
    nt = pl.num_programs(1)

    logits = _dot_t(x_ref[...], gate_ref[...])
    mask1, mask2, w1, w2 = _top2(logits)
    onehot = jnp.where(jnp.logical_or(mask1, mask2), 1.0, 0.0)

    @pl.when(jnp.logical_and(p == 0, i == 0))
    def _():
        st_ref[0:1, :] = jnp.zeros_like(st_ref[0:1, :])

    @pl.when(p == 0)
    def _():
        st_ref[0:1, 0:E] += jnp.sum(onehot, axis=0, keepdims=True)

    @pl.when(jnp.logical_and(p == 0, i == nt - 1))
    def _():
        cnt = st_ref[0:1, 0:E].astype(jnp.int32)
        padded = ((cnt + TMG - 1) // TMG) * TMG
        padded_f = padded.astype(jnp.float32)
        r8 = jax.lax.broadcasted_iota(jnp.int32, (E, E), 0)
        c8 = jax.lax.broadcasted_iota(jnp.int32, (E, E), 1)
        tri_lt = jnp.where(r8 < c8, 1.0, 0.0)
        tri_le = jnp.where(r8 <= c8, 1.0, 0.0)
        offs = jax.lax.dot_general(padded_f, tri_lt, (((1,), (0,)), ((), ())),
                                   preferred_element_type=jnp.float32)
        cum = jax.lax.dot_general(padded_f, tri_le, (((1,), (0,)), ((), ())),
                                  preferred_element_type=jnp.float32)
        st_ref[1:2, 0:E] = offs
        jlane = jax.lax.broadcasted_iota(jnp.int32, (1, 128), 1) * TMG
        te = jnp.zeros((1, 128), jnp.int32)
        for e in range(E):
            ce = cum[0, e].astype(jnp.int32)
            te = te + jnp.where(ce <= jlane, 1, 0)
        te = jnp.minimum(te, E - 1)
        meta_ref[...] = jnp.broadcast_to(te, meta_ref.shape)

    @pl.when(jnp.logical_and(p == 1, i == 0))
    def _():
        st_ref[3:4, :] = jnp.zeros_like(st_ref[3:4, :])

    @pl.when(p == 1)
    def _():
        rr = jax.lax.broadcasted_iota(jnp.int32, (TM, TM), 0)
        cc = jax.lax.broadcasted_iota(jnp.int32, (TM, TM), 1)
        tri = jnp.where(rr > cc, 1.0, 0.0)
        rank = jax.lax.dot_general(tri, onehot, (((1,), (0,)), ((), ())),
                                   preferred_element_type=jnp.float32)
        rank = rank + st_ref[3:4, 0:E]
        st_ref[3:4, 0:E] += jnp.sum(onehot, axis=0, keepdims=True)
        posv = rank + st_ref[1:2, 0:E]
        pos0 = jnp.sum(jnp.where(mask1, posv, 0.0), axis=1, keepdims=True)
        pos1 = jnp.sum(jnp.where(mask2, posv, 0.0), axis=1, keepdims=True)
        col = jax.lax.broadcasted_iota(jnp.int32, (TM, E), 1)
        pos_ref[...] = (jnp.where(col == 0, pos0, 0.0)
                        + jnp.where(col == 1, pos1, 0.0)).astype(jnp.int32)
        w_ref[...] = (jnp.where(col == 0, w1, 0.0)
                      + jnp.where(col == 1, w2, 0.0))


def _shared_kernel(x_ref, gu_g_ref, gu_u_ref, down_ref, sh_out_ref, acc_ref):
    c = pl.program_id(1)

    @pl.when(c == 0)
    def _():
        acc_ref[...] = jnp.zeros_like(acc_ref)

    x = x_ref[...]
    g = _dot_t(x, gu_g_ref[...])
    u = _dot_t(x, gu_u_ref[...])
    h = _silu(g) * u
    acc_ref[...] += _dot_t(h, down_ref[...])

    @pl.when(c == pl.num_programs(1) - 1)
    def _():
        sh_out_ref[...] = acc_ref[...]


def _group_kernel(te_ref, xs_ref, wg_ref, wu_ref, wd_ref, y_ref, acc_ref):
    c = pl.program_id(1)

    @pl.when(c == 0)
    def _():
        acc_ref[...] = jnp.zeros_like(acc_ref)

    xb = xs_ref[...]
    g = _dot_t(xb, wg_ref[0])
    u = _dot_t(xb, wu_ref[0])
    h = _silu(g) * u
    acc_ref[...] += _dot_t(h, wd_ref[0])

    @pl.when(c == pl.num_programs(1) - 1)
    def _():
        y_ref[...] = acc_ref[...]


def _sc_dispatch_body(x_hbm, posf_hbm, xs_hbm, pos_v, idx_v, xbuf, sem):
    cid = jax.lax.axis_index("c")
    sid = jax.lax.axis_index("s")
    wid = sid * 2 + cid
    base = wid * TPW
    pltpu.sync_copy(posf_hbm.at[pl.ds(base * E, TPW * E)], pos_v)
    iota = jax.lax.iota(jnp.int32, 16)
    nch = TPW // 16
    for ch in range(nch):
        flat = (iota + ch * 16) * E
        idx_v[2 * ch] = plsc.load_gather(pos_v, [flat])
        idx_v[2 * ch + 1] = plsc.load_gather(pos_v, [flat + 1])
    for ch in range(nch):
        tb = base + ch * 16
        pltpu.sync_copy(x_hbm.at[pl.ds(tb, 16)], xbuf)
        cp0 = pltpu.async_copy(xbuf, xs_hbm.at[idx_v.at[2 * ch]], sem)
        cp0.wait()
        cp1 = pltpu.async_copy(xbuf, xs_hbm.at[idx_v.at[2 * ch + 1]], sem)
        cp1.wait()


def _sc_combine_body(y_hbm, sh_hbm, posf_hbm, wf_hbm, out_hbm,
                     pos_v, w_v, idx_v, y0, y1, acc, sem):
    cid = jax.lax.axis_index("c")
    sid = jax.lax.axis_index("s")
    wid = sid * 2 + cid
    base = wid * TPW
    pltpu.sync_copy(posf_hbm.at[pl.ds(base * E, TPW * E)], pos_v)
    pltpu.sync_copy(wf_hbm.at[pl.ds(base * E, TPW * E)], w_v)
    iota = jax.lax.iota(jnp.int32, 16)
    zeros = jnp.zeros((16,), jnp.int32)
    nch = TPW // 16
    for ch in range(nch):
        flat = (iota + ch * 16) * E
        idx_v[2 * ch] = plsc.load_gather(pos_v, [flat])
        idx_v[2 * ch + 1] = plsc.load_gather(pos_v, [flat + 1])
    for ch in range(nch):
        tb = base + ch * 16
        cp0 = pltpu.async_copy(y_hbm.at[idx_v.at[2 * ch]], y0, sem)
        cp1 = pltpu.async_copy(y_hbm.at[idx_v.at[2 * ch + 1]], y1, sem)
        pltpu.sync_copy(sh_hbm.at[pl.ds(tb, 16)], acc)
        cp0.wait()
        cp1.wait()

        def tok_body(t, carry):
            fidx = zeros + (ch * 16 + t) * E
            w0 = plsc.load_gather(w_v, [fidx])
            w1 = plsc.load_gather(w_v, [fidx + 1])

            def q_body(q, c2):
                sl = pl.ds(q * 16, 16)
                acc[t, sl] = acc[t, sl] + w0 * y0[t, sl] + w1 * y1[t, sl]
                return c2

            return jax.lax.fori_loop(0, H // 16, q_body, carry)

        jax.lax.fori_loop(0, 16, tok_body, 0)
        pltpu.sync_copy(acc, out_hbm.at[pl.ds(tb, 16)])


@functools.lru_cache(maxsize=None)
def _sc_kernels():
    mesh = plsc.VectorSubcoreMesh(core_axis_name="c", subcore_axis_name="s")
    params = pltpu.CompilerParams(needs_layout_passes=False)
    dispatch = pl.kernel(
        _sc_dispatch_body,
        out_type=jax.ShapeDtypeStruct((L, H), jnp.float32),
        mesh=mesh,
        compiler_params=params,
        scratch_types=[
            pltpu.VMEM((TPW * E,), jnp.int32),
            pltpu.VMEM((2 * (TPW // 16), 16), jnp.int32),
            pltpu.VMEM((16, H), jnp.float32),
            pltpu.SemaphoreType.DMA,
        ],
    )
    combine = pl.kernel(
        _sc_combine_body,
        out_type=jax.ShapeDtypeStruct((T, H), jnp.float32),
        mesh=mesh,
        compiler_params=params,
        scratch_types=[
            pltpu.VMEM((TPW * E,), jnp.int32),
            pltpu.VMEM((TPW * E,), jnp.float32),
            pltpu.VMEM((2 * (TPW // 16), 16), jnp.int32),
            pltpu.VMEM((16, H), jnp.float32),
            pltpu.VMEM((16, H), jnp.float32),
            pltpu.VMEM((16, H), jnp.float32),
            pltpu.SemaphoreType.DMA,
        ],
    )
    return dispatch, combine


def _sc_dispatch(x, pos):
    return _sc_kernels()[0](x, pos.reshape(-1))


def _sc_combine(y, sh, pos, w):
    return _sc_kernels()[1](y, sh, pos.reshape(-1), w.reshape(-1))


def _router(x, gate_w):
    return pl.pallas_call(
        _router_kernel,
        grid=(2, T // TM),
        in_specs=[
            pl.BlockSpec((TM, H), lambda p, i: (i, 0)),
            pl.BlockSpec((E, H), lambda p, i: (0, 0)),
        ],
        out_specs=[
            pl.BlockSpec((TM, E), lambda p, i: (i, 0)),
            pl.BlockSpec((TM, E), lambda p, i: (i, 0)),
            pl.BlockSpec((8, 128), lambda p, i: (0, 0)),
        ],
        out_shape=[
            jax.ShapeDtypeStruct((T, E), jnp.int32),
            jax.ShapeDtypeStruct((T, E), jnp.float32),
            jax.ShapeDtypeStruct((8, 128), jnp.int32),
        ],
        scratch_shapes=[pltpu.VMEM((8, 128), jnp.float32)],
    )(x, gate_w)


def _shared(x, shared_gate_up_w, shared_down_w):
    return pl.pallas_call(
        _shared_kernel,
        grid=(T // TM, SH // KC),
        in_specs=[
            pl.BlockSpec((TM, H), lambda i, c: (i, 0)),
            pl.BlockSpec((KC, H), lambda i, c: (c, 0)),
            pl.BlockSpec((KC, H), lambda i, c: (SH // KC + c, 0)),
            pl.BlockSpec((H, KC), lambda i, c: (0, c)),
        ],
        out_specs=pl.BlockSpec((TM, H), lambda i, c: (i, 0)),
        out_shape=jax.ShapeDtypeStruct((T, H), jnp.float32),
        scratch_shapes=[pltpu.VMEM((TM, H), jnp.float32)],
    )(x, shared_gate_up_w, shared_gate_up_w, shared_down_w)


def _grouped(te, xs, w_gate, w_up, w_down):
    grid_spec = pltpu.PrefetchScalarGridSpec(
        num_scalar_prefetch=1,
        grid=(NTILES, DFF // KC),
        in_specs=[
            pl.BlockSpec((TMG, H), lambda i, c, te_r: (i, 0)),
            pl.BlockSpec((1, KC, H), lambda i, c, te_r: (te_r[i], c, 0)),
            pl.BlockSpec((1, KC, H), lambda i, c, te_r: (te_r[i], c, 0)),
            pl.BlockSpec((1, H, KC), lambda i, c, te_r: (te_r[i], 0, c)),
        ],
        out_specs=pl.BlockSpec((TMG, H), lambda i, c, te_r: (i, 0)),
        scratch_shapes=[pltpu.VMEM((TMG, H), jnp.float32)],
    )
    return pl.pallas_call(
        _group_kernel,
        grid_spec=grid_spec,
        out_shape=jax.ShapeDtypeStruct((L, H), jnp.float32),
    )(te, xs, w_gate, w_up, w_down)


def kernel(hidden_states, gate_w, w_gate, w_up, w_down, shared_gate_up_w,
           shared_down_w):
    orig_shape = hidden_states.shape
    x = hidden_states.reshape(-1, H)

    pos, w, meta = _router(x, gate_w)
    te = meta[0, :NTILES]
    xs = _sc_dispatch(x, pos)
    sh = _shared(x, shared_gate_up_w, shared_down_w)
    y = _grouped(te, xs, w_gate, w_up, w_down)
    out = _sc_combine(y, sh, pos, w)
    return out.reshape(orig_shape)


# speedup vs baseline: 1.7956x; 1.0708x over previous
import functools

import jax
import jax.numpy as jnp
from jax.experimental import pallas as pl
from jax.experimental.pallas import tpu as pltpu
from jax.experimental.pallas import tpu_sc as plsc

H = 2048
E = 8
DFF = 2048
SH = 2048
T = 2048
TOPK = 2

TM = 256
TMS = 256
TMG = 256
L = ((T * TOPK + E * (TMG - 1)) + TMG - 1) // TMG * TMG
NTILES = L // TMG
NEG_INF = -1e30

NW = 32
TPW = T // NW


def _silu(v):
    return v * (1.0 / (1.0 + jnp.exp(-v)))


def _dot_t(a, b):
    return jax.lax.dot_general(a, b, (((1,), (1,)), ((), ())),
                               preferred_element_type=jnp.float32)


def _top2(logits):
    col = jax.lax.broadcasted_iota(jnp.int32, logits.shape, 1)
    m1 = jnp.max(logits, axis=1, keepdims=True)
    a1 = jnp.min(jnp.where(logits == m1, col, E), axis=1, keepdims=True)
    mask1 = col == a1
    l2 = jnp.where(mask1, NEG_INF, logits)
    m2 = jnp.max(l2, axis=1, keepdims=True)
    a2 = jnp.min(jnp.where(l2 == m2, col, E), axis=1, keepdims=True)
    mask2 = col == a2
    w1 = 1.0 / (1.0 + jnp.exp(m2 - m1))
    w2 = 1.0 - w1
    return mask1, mask2, w1, w2


def _router_kernel(x_ref, gate_ref, pos_ref, w_ref, meta_ref, st_ref):
    p = pl.program_id(0)
    i = pl.program_id(1)
    nt = pl.num_programs(1)

    logits = _dot_t(x_ref[...], gate_ref[...])
    mask1, mask2, w1, w2 = _top2(logits)
    onehot = jnp.where(jnp.logical_or(mask1, mask2), 1.0, 0.0)

    @pl.when(jnp.logical_and(p == 0, i == 0))
    def _():
        st_ref[0:1, :] = jnp.zeros_like(st_ref[0:1, :])

    @pl.when(p == 0)
    def _():
        st_ref[0:1, 0:E] += jnp.sum(onehot, axis=0, keepdims=True)

    @pl.when(jnp.logical_and(p == 0, i == nt - 1))
    def _():
        cnt = st_ref[0:1, 0:E].astype(jnp.int32)
        padded = ((cnt + TMG - 1) // TMG) * TMG
        padded_f = padded.astype(jnp.float32)
        r8 = jax.lax.broadcasted_iota(jnp.int32, (E, E), 0)
        c8 = jax.lax.broadcasted_iota(jnp.int32, (E, E), 1)
        tri_lt = jnp.where(r8 < c8, 1.0, 0.0)
        tri_le = jnp.where(r8 <= c8, 1.0, 0.0)
        offs = jax.lax.dot_general(padded_f, tri_lt, (((1,), (0,)), ((), ())),
                                   preferred_element_type=jnp.float32)
        cum = jax.lax.dot_general(padded_f, tri_le, (((1,), (0,)), ((), ())),
                                  preferred_element_type=jnp.float32)
        st_ref[1:2, 0:E] = offs
        jlane = jax.lax.broadcasted_iota(jnp.int32, (1, 128), 1) * TMG
        te = jnp.zeros((1, 128), jnp.int32)
        for e in range(E):
            ce = cum[0, e].astype(jnp.int32)
            te = te + jnp.where(ce <= jlane, 1, 0)
        te = jnp.minimum(te, E - 1)
        meta_ref[...] = jnp.broadcast_to(te, meta_ref.shape)

    @pl.when(jnp.logical_and(p == 1, i == 0))
    def _():
        st_ref[3:4, :] = jnp.zeros_like(st_ref[3:4, :])

    @pl.when(p == 1)
    def _():
        rr = jax.lax.broadcasted_iota(jnp.int32, (TM, TM), 0)
        cc = jax.lax.broadcasted_iota(jnp.int32, (TM, TM), 1)
        tri = jnp.where(rr > cc, 1.0, 0.0)
        rank = jax.lax.dot_general(tri, onehot, (((1,), (0,)), ((), ())),
                                   preferred_element_type=jnp.float32)
        rank = rank + st_ref[3:4, 0:E]
        st_ref[3:4, 0:E] += jnp.sum(onehot, axis=0, keepdims=True)
        posv = rank + st_ref[1:2, 0:E]
        pos0 = jnp.sum(jnp.where(mask1, posv, 0.0), axis=1, keepdims=True)
        pos1 = jnp.sum(jnp.where(mask2, posv, 0.0), axis=1, keepdims=True)
        col = jax.lax.broadcasted_iota(jnp.int32, (TM, E), 1)
        pos_ref[...] = (jnp.where(col == 0, pos0, 0.0)
                        + jnp.where(col == 1, pos1, 0.0)).astype(jnp.int32)
        w_ref[...] = (jnp.where(col == 0, w1, 0.0)
                      + jnp.where(col == 1, w2, 0.0))


def _shared_kernel(x_ref, gu_g_ref, gu_u_ref, down_ref, sh_out_ref):
    xb = x_ref[...].astype(jnp.bfloat16)
    g = _dot_t(xb, gu_g_ref[0])
    u = _dot_t(xb, gu_u_ref[0])
    h = (_silu(g) * u).astype(jnp.bfloat16)
    sh_out_ref[...] = _dot_t(h, down_ref[...])


def _group_kernel(te_ref, xs_ref, wg_ref, wu_ref, wd_ref, y_ref):
    v = xs_ref[...]
    lo = jax.lax.bitcast_convert_type(v << 16, jnp.float32)
    hi = jax.lax.bitcast_convert_type(
        v & jnp.int32(-65536), jnp.float32)
    xb = jnp.concatenate([lo, hi], axis=1).astype(jnp.bfloat16)
    g = _dot_t(xb, wg_ref[0])
    u = _dot_t(xb, wu_ref[0])
    h = (_silu(g) * u).astype(jnp.bfloat16)
    y_ref[...] = _dot_t(h, wd_ref[0])


def _sc_dispatch_body(x_hbm, posf_hbm, xs_hbm, pos_v, idx_v, xbuf, semin,
                      semout):
    cid = jax.lax.axis_index("c")
    sid = jax.lax.axis_index("s")
    wid = sid * 2 + cid
    base = wid * TPW
    pltpu.sync_copy(posf_hbm.at[pl.ds(base * E, TPW * E)], pos_v)
    iota = jax.lax.iota(jnp.int32, 16)
    nch = TPW // 16
    for ch in range(nch):
        flat = (iota + ch * 16) * E
        idx_v[2 * ch] = plsc.load_gather(pos_v, [flat])
        idx_v[2 * ch + 1] = plsc.load_gather(pos_v, [flat + 1])
    ins = []
    for ch in range(nch):
        tb = base + ch * 16
        ins.append(pltpu.async_copy(x_hbm.at[pl.ds(tb, 16)], xbuf.at[ch],
                                    semin))
    scats = []
    for ch in range(nch):
        ins[ch].wait()
        scats.append(pltpu.async_copy(xbuf.at[ch],
                                      xs_hbm.at[idx_v.at[2 * ch]], semout))
        scats.append(pltpu.async_copy(xbuf.at[ch],
                                      xs_hbm.at[idx_v.at[2 * ch + 1]],
                                      semout))
    for cp in scats:
        cp.wait()


def _sc_combine_body(y_hbm, sh_hbm, posf_hbm, wf_hbm, out_hbm,
                     pos_v, w_v, idx_v, yb, acc, semg, semin, semout):
    cid = jax.lax.axis_index("c")
    sid = jax.lax.axis_index("s")
    wid = sid * 2 + cid
    base = wid * TPW
    pltpu.sync_copy(posf_hbm.at[pl.ds(base * E, TPW * E)], pos_v)
    pltpu.sync_copy(wf_hbm.at[pl.ds(base * E, TPW * E)], w_v)
    iota = jax.lax.iota(jnp.int32, 16)
    zeros = jnp.zeros((16,), jnp.int32)
    tok8 = jnp.where(iota < 8, iota, iota - 8)
    slot = jnp.where(iota < 8, 0, 1)
    nch = TPW // 8
    for ch in range(nch):
        flat = (tok8 + ch * 8) * E + slot
        idx_v[ch] = plsc.load_gather(pos_v, [flat])

    gets = [None] * nch
    shs = [None] * nch
    outs = [None] * nch

    def _issue(ch):
        tb = base + ch * 8
        gets[ch] = pltpu.async_copy(y_hbm.at[idx_v.at[ch]], yb.at[ch % 2],
                                    semg)
        shs[ch] = pltpu.async_copy(sh_hbm.at[pl.ds(tb, 8)], acc.at[ch % 2],
                                   semin)

    _issue(0)
    for ch in range(nch):
        if ch + 1 < nch:
            if ch >= 1:
                outs[ch - 1].wait()
            _issue(ch + 1)
        gets[ch].wait()
        shs[ch].wait()
        b = ch % 2

        def tok_body(t, carry):
            fidx = zeros + (ch * 8 + t) * E
            w0 = plsc.load_gather(w_v, [fidx])
            w1 = plsc.load_gather(w_v, [fidx + 1])

            def q_body(q, c2):
                for u in range(4):
                    sl = pl.ds(q * 64 + u * 16, 16)
                    acc[b, t, sl] = (acc[b, t, sl] + w0 * yb[b, t, sl]
                                     + w1 * yb[b, t + 8, sl])
                return c2

            return jax.lax.fori_loop(0, H // 64, q_body, carry)

        jax.lax.fori_loop(0, 8, tok_body, 0)
        outs[ch] = pltpu.async_copy(acc.at[b], out_hbm.at[pl.ds(base + ch * 8, 8)],
                                    semout)
    outs[nch - 1].wait()
    outs[nch - 2].wait()


@functools.lru_cache(maxsize=None)
def _sc_kernels():
    mesh = plsc.VectorSubcoreMesh(core_axis_name="c", subcore_axis_name="s")
    params = pltpu.CompilerParams(needs_layout_passes=False)
    dispatch = pl.kernel(
        _sc_dispatch_body,
        out_type=jax.ShapeDtypeStruct((L, H // 2), jnp.int32),
        mesh=mesh,
        compiler_params=params,
        scratch_types=[
            pltpu.VMEM((TPW * E,), jnp.int32),
            pltpu.VMEM((2 * (TPW // 16), 16), jnp.int32),
            pltpu.VMEM((TPW // 16, 16, H // 2), jnp.int32),
            pltpu.SemaphoreType.DMA,
            pltpu.SemaphoreType.DMA,
        ],
    )
    combine = pl.kernel(
        _sc_combine_body,
        out_type=jax.ShapeDtypeStruct((T, H), jnp.float32),
        mesh=mesh,
        compiler_params=params,
        scratch_types=[
            pltpu.VMEM((TPW * E,), jnp.int32),
            pltpu.VMEM((TPW * E,), jnp.float32),
            pltpu.VMEM((TPW // 8, 16), jnp.int32),
            pltpu.VMEM((2, 16, H), jnp.float32),
            pltpu.VMEM((2, 8, H), jnp.float32),
            pltpu.SemaphoreType.DMA,
            pltpu.SemaphoreType.DMA,
            pltpu.SemaphoreType.DMA,
        ],
    )
    return dispatch, combine


def _sc_dispatch(x16, pos):
    return _sc_kernels()[0](x16, pos.reshape(-1))


def _sc_combine(y, sh, pos, w):
    return _sc_kernels()[1](y, sh, pos.reshape(-1), w.reshape(-1))


def _router(x, gate_w):
    return pl.pallas_call(
        _router_kernel,
        grid=(2, T // TM),
        in_specs=[
            pl.BlockSpec((TM, H), lambda p, i: (i, 0)),
            pl.BlockSpec((E, H), lambda p, i: (0, 0)),
        ],
        out_specs=[
            pl.BlockSpec((TM, E), lambda p, i: (i, 0)),
            pl.BlockSpec((TM, E), lambda p, i: (i, 0)),
            pl.BlockSpec((8, 128), lambda p, i: (0, 0)),
        ],
        out_shape=[
            jax.ShapeDtypeStruct((T, E), jnp.int32),
            jax.ShapeDtypeStruct((T, E), jnp.float32),
            jax.ShapeDtypeStruct((8, 128), jnp.int32),
        ],
        scratch_shapes=[pltpu.VMEM((8, 128), jnp.float32)],
    )(x, gate_w)


def _shared(x, gu2, down_w):
    return pl.pallas_call(
        _shared_kernel,
        grid=(T // TMS,),
        in_specs=[
            pl.BlockSpec((TMS, H), lambda i: (i, 0)),
            pl.BlockSpec((1, SH, H), lambda i: (0, 0, 0)),
            pl.BlockSpec((1, SH, H), lambda i: (1, 0, 0)),
            pl.BlockSpec((H, SH), lambda i: (0, 0)),
        ],
        out_specs=pl.BlockSpec((TMS, H), lambda i: (i, 0)),
        out_shape=jax.ShapeDtypeStruct((T, H), jnp.float32),
    )(x, gu2, gu2, down_w)


def _grouped(te, xs, w_gate, w_up, w_down):
    grid_spec = pltpu.PrefetchScalarGridSpec(
        num_scalar_prefetch=1,
        grid=(NTILES,),
        in_specs=[
            pl.BlockSpec((TMG, H // 2), lambda i, te_r: (i, 0)),
            pl.BlockSpec((1, DFF, H), lambda i, te_r: (te_r[i], 0, 0)),
            pl.BlockSpec((1, DFF, H), lambda i, te_r: (te_r[i], 0, 0)),
            pl.BlockSpec((1, H, DFF), lambda i, te_r: (te_r[i], 0, 0)),
        ],
        out_specs=pl.BlockSpec((TMG, H), lambda i, te_r: (i, 0)),
    )
    return pl.pallas_call(
        _group_kernel,
        grid_spec=grid_spec,
        out_shape=jax.ShapeDtypeStruct((L, H), jnp.float32),
    )(te, xs, w_gate, w_up, w_down)


def kernel(hidden_states, gate_w, w_gate, w_up, w_down, shared_gate_up_w,
           shared_down_w):
    orig_shape = hidden_states.shape
    x = hidden_states.reshape(-1, H)

    gu16 = shared_gate_up_w.astype(jnp.bfloat16).reshape(2, SH, H)
    down16 = shared_down_w.astype(jnp.bfloat16)
    wg16 = w_gate.astype(jnp.bfloat16)
    wu16 = w_up.astype(jnp.bfloat16)
    wd16 = w_down.astype(jnp.bfloat16)
    x16 = x.astype(jnp.bfloat16)
    xi = jax.lax.bitcast_convert_type(
        jnp.stack([x16[:, :H // 2], x16[:, H // 2:]], axis=-1), jnp.int32)

    pos, w, meta = _router(x, gate_w)
    te = meta[0, :NTILES]
    xs = _sc_dispatch(xi, pos)
    sh = _shared(x, gu16, down16)
    y = _grouped(te, xs, wg16, wu16, wd16)
    out = _sc_combine(y, sh, pos, w)
    return out.reshape(orig_shape)
